# baseline (speedup 1.0000x reference)
import functools

import jax
import jax.numpy as jnp
from jax.experimental import pallas as pl
from jax.experimental.pallas import tpu as pltpu

_M, _N, _BC = 128, 100000, 4096
_NB = pl.cdiv(_N, _BC)
_EPS = 1e-20


@functools.cache
def _gumbel_noise_blocked():
    nkey = jax.random.key(42)
    u = jax.random.uniform(nkey, (_M, _N), dtype=jnp.float32)
    z = -jnp.log(-jnp.log(u + _EPS) + _EPS)
    zp = jnp.pad(z, ((0, 0), (0, _NB * _BC - _N)))
    return zp.reshape(_M, _NB, _BC).transpose(1, 0, 2)


def _add_kernel(x_ref, z_ref, o_ref):
    o_ref[...] = x_ref[...] + z_ref[0]


def kernel(dist):
    z3 = _gumbel_noise_blocked()
    return pl.pallas_call(
        _add_kernel,
        grid=(_NB,),
        in_specs=[
            pl.BlockSpec((_M, _BC), lambda j: (0, j)),
            pl.BlockSpec((1, _M, _BC), lambda j: (j, 0, 0)),
        ],
        out_specs=pl.BlockSpec((_M, _BC), lambda j: (0, j)),
        out_shape=jax.ShapeDtypeStruct((_M, _N), jnp.float32),
        compiler_params=pltpu.CompilerParams(dimension_semantics=("arbitrary",)),
    )(dist, z3)


# speedup vs baseline: 1.1191x; 1.1191x over previous
import jax
import jax.numpy as jnp
import numpy as np
from jax.experimental import pallas as pl
from jax.experimental.pallas import tpu as pltpu

_M, _N = 128, 100000
_BC = 4096
_NB = pl.cdiv(_N, _BC)
_EPS = 1e-20

_K0 = np.uint32(0)
_K1 = np.uint32(42)
_K2 = np.uint32(0 ^ 42 ^ 0x1BD11BDA)
_ROT_A = (13, 15, 26, 6)
_ROT_B = (17, 29, 16, 24)


def _rotl(x, d):
    return (x << np.uint32(d)) | (x >> np.uint32(32 - d))


def _threefry_bits(cnt):
    x0 = jnp.zeros_like(cnt) + _K0
    x1 = cnt + _K1
    ks = (_K0, _K1, _K2)
    for inj in range(5):
        rots = _ROT_A if inj % 2 == 0 else _ROT_B
        for r in rots:
            x0 = x0 + x1
            x1 = _rotl(x1, r)
            x1 = x0 ^ x1
        x0 = x0 + ks[(inj + 1) % 3]
        x1 = x1 + ks[(inj + 2) % 3] + np.uint32(inj + 1)
    return x0 ^ x1


def _gumbel_argmax_kernel(dist_ref, idx_ref, m_scr, i_scr):
    j = pl.program_id(0)
    row = jax.lax.broadcasted_iota(jnp.int32, (_M, _BC), 0)
    col = j * _BC + jax.lax.broadcasted_iota(jnp.int32, (_M, _BC), 1)
    cnt = (row * _N + col).astype(jnp.uint32)
    bits = _threefry_bits(cnt)
    fbits = (bits >> np.uint32(9)) | np.uint32(0x3F800000)
    u = jax.lax.bitcast_convert_type(fbits, jnp.float32) - jnp.float32(1.0)
    t = jnp.log(u + _EPS)
    z = -jnp.log(_EPS - t)
    d = dist_ref[...] + z
    d = jnp.where(col < _N, d, -jnp.inf)
    bm = jnp.max(d, axis=1, keepdims=True)
    bi = jnp.min(jnp.where(d == bm, col, _N), axis=1, keepdims=True)

    @pl.when(j == 0)
    def _():
        m_scr[...] = bm
        i_scr[...] = bi

    @pl.when(j != 0)
    def _():
        better = bm > m_scr[...]
        i_scr[...] = jnp.where(better, bi, i_scr[...])
        m_scr[...] = jnp.where(better, bm, m_scr[...])

    @pl.when(j == _NB - 1)
    def _():
        idx_ref[...] = i_scr[...]


def _onehot_kernel(idx_ref, out_ref):
    j = pl.program_id(0)
    col = j * _BC + jax.lax.broadcasted_iota(jnp.int32, (_M, _BC), 1)
    out_ref[...] = jnp.where(col == idx_ref[...],
                             jnp.float32(1.0), jnp.float32(0.0))


def kernel(dist):
    idx = pl.pallas_call(
        _gumbel_argmax_kernel,
        grid=(_NB,),
        in_specs=[pl.BlockSpec((_M, _BC), lambda j: (0, j))],
        out_specs=pl.BlockSpec((_M, 1), lambda j: (0, 0)),
        out_shape=jax.ShapeDtypeStruct((_M, 1), jnp.int32),
        scratch_shapes=[
            pltpu.VMEM((_M, 1), jnp.float32),
            pltpu.VMEM((_M, 1), jnp.int32),
        ],
        compiler_params=pltpu.CompilerParams(
            dimension_semantics=("arbitrary",),
        ),
    )(dist)
    return pl.pallas_call(
        _onehot_kernel,
        grid=(_NB,),
        in_specs=[pl.BlockSpec((_M, 1), lambda j: (0, 0))],
        out_specs=pl.BlockSpec((_M, _BC), lambda j: (0, j)),
        out_shape=jax.ShapeDtypeStruct((_M, _N), jnp.float32),
        compiler_params=pltpu.CompilerParams(
            dimension_semantics=("arbitrary",),
        ),
    )(idx)
